# baseline (speedup 1.0000x reference)
import jax
import jax.numpy as jnp
from jax.experimental import pallas as pl
from jax.experimental.pallas import tpu as pltpu


def _film_body(t_ref, x_ref, emb_ref, o_ref):
    shift = emb_ref[0, 0, :]
    scale = emb_ref[0, 1, :]
    o_ref[...] = jax.nn.gelu(x_ref[...] * scale + shift)


def kernel(x, timestep, film_table):
    B, S, D = x.shape
    S_BLK = 1024
    table3 = film_table.reshape(film_table.shape[0], 2, D)
    grid = (B, S // S_BLK)
    out = pl.pallas_call(
        _film_body,
        grid_spec=pltpu.PrefetchScalarGridSpec(
            num_scalar_prefetch=1,
            grid=grid,
            in_specs=[
                pl.BlockSpec((1, S_BLK, D), lambda b, s, t_ref: (b, s, 0)),
                pl.BlockSpec((1, 2, D), lambda b, s, t_ref: (t_ref[b], 0, 0)),
            ],
            out_specs=pl.BlockSpec((1, S_BLK, D), lambda b, s, t_ref: (b, s, 0)),
        ),
        out_shape=jax.ShapeDtypeStruct((B, S, D), x.dtype),
    )(timestep, x, table3)
    return out
